# baseline (speedup 1.0000x reference)
import functools

import jax
import jax.numpy as jnp
from jax import lax
from jax.experimental import pallas as pl
from jax.experimental.pallas import tpu as pltpu
from jax.experimental.pallas import tpu_sc as plsc

N = 10000
H = 128
E = 320000

NC = 2
NS = 16
NW = NC * NS

N2 = 10240
RPW = N2 // NW
RPS = N2 // NS
CS = 128
CH_CORE = (112, 48)
CHMAX = max(CH_CORE)
E0 = NS * CH_CORE[0] * CS
E1 = NS * CH_CORE[1] * CS
EPAD = E0 + E1

_mesh = plsc.VectorSubcoreMesh(
    core_axis_name="c", subcore_axis_name="s", num_cores=NC, num_subcores=NS)


CH_CNT = (EPAD // (2 * NS * CS), EPAD // (2 * NS * CS))


def _prep_body(emb_h, nid_h, d0_h, d1_h, zr_h, ones_h, x0_h, pc_h,
               nid_v, rows_v, didx_v, ones_v, cnt_sh):
    c = lax.axis_index("c")
    s = lax.axis_index("s")
    pltpu.sync_copy(nid_h.at[c, s], nid_v)
    pltpu.sync_copy(ones_h, ones_v)
    pltpu.sync_copy(zr_h, cnt_sh.at[pl.ds(s * RPS, RPS)])

    @pl.when(c == 0)
    def _():
        pltpu.sync_copy(d0_h.at[s], didx_v.at[pl.ds(0, CH_CNT[0])])

    @pl.when(c == 1)
    def _():
        pltpu.sync_copy(d1_h.at[s], didx_v.at[pl.ds(0, CH_CNT[1])])

    plsc.subcore_barrier()
    w = c * NS + s
    for j in range(5):
        pltpu.sync_copy(emb_h.at[nid_v.at[j]], rows_v)
        pltpu.sync_copy(rows_v, x0_h.at[pl.ds(w * RPW + j * 64, 64)])

    def count_loop(nch):
        def body(k, _):
            pltpu.sync_copy(ones_v, cnt_sh.at[didx_v.at[k]], add=True)
            return 0
        lax.fori_loop(0, nch, body, 0)

    @pl.when(c == 0)
    def _():
        count_loop(CH_CNT[0])

    @pl.when(c == 1)
    def _():
        count_loop(CH_CNT[1])

    plsc.subcore_barrier()
    pltpu.sync_copy(cnt_sh.at[pl.ds(s * RPS, RPS)],
                    pc_h.at[c, pl.ds(s * RPS, RPS)])


_prep_kernel = pl.kernel(
    _prep_body,
    out_type=(jax.ShapeDtypeStruct((N2, H), jnp.float32),
              jax.ShapeDtypeStruct((NC, N2, H), jnp.float32)),
    mesh=_mesh,
    scratch_types=[
        pltpu.VMEM((5, 64), jnp.int32),
        pltpu.VMEM((64, H), jnp.float32),
        pltpu.VMEM((CHMAX, CS), jnp.int32),
        pltpu.VMEM((CS, H), jnp.float32),
        pltpu.VMEM_SHARED((N2, H), jnp.float32),
    ],
)


def _seg_body(x_h, s0_h, d0_h, s1_h, d1_h, zr_h, p_h,
              sidx_v, didx_v, rows_v, acc_sh):
    c = lax.axis_index("c")
    s = lax.axis_index("s")
    pltpu.sync_copy(zr_h, acc_sh.at[pl.ds(s * RPS, RPS)])

    @pl.when(c == 0)
    def _():
        pltpu.sync_copy(s0_h.at[s], sidx_v.at[pl.ds(0, CH_CORE[0])])
        pltpu.sync_copy(d0_h.at[s], didx_v.at[pl.ds(0, CH_CORE[0])])

    @pl.when(c == 1)
    def _():
        pltpu.sync_copy(s1_h.at[s], sidx_v.at[pl.ds(0, CH_CORE[1])])
        pltpu.sync_copy(d1_h.at[s], didx_v.at[pl.ds(0, CH_CORE[1])])

    plsc.subcore_barrier()

    def edge_loop(nch):
        def body(k, _):
            pltpu.sync_copy(x_h.at[sidx_v.at[k]], rows_v)
            pltpu.sync_copy(rows_v, acc_sh.at[didx_v.at[k]], add=True)
            return 0
        lax.fori_loop(0, nch, body, 0)

    @pl.when(c == 0)
    def _():
        edge_loop(CH_CORE[0])

    @pl.when(c == 1)
    def _():
        edge_loop(CH_CORE[1])

    plsc.subcore_barrier()
    pltpu.sync_copy(acc_sh.at[pl.ds(s * RPS, RPS)],
                    p_h.at[c, pl.ds(s * RPS, RPS)])


_seg_kernel = pl.kernel(
    _seg_body,
    out_type=jax.ShapeDtypeStruct((NC, N2, H), jnp.float32),
    mesh=_mesh,
    scratch_types=[
        pltpu.VMEM((CHMAX, CS), jnp.int32),
        pltpu.VMEM((CHMAX, CS), jnp.int32),
        pltpu.VMEM((CS, H), jnp.float32),
        pltpu.VMEM_SHARED((N2, H), jnp.float32),
    ],
)

BLK = 1280


def _tc_body(p_ref, pc_ref, x_ref, wl_ref, wr_ref, bl_ref, o_ref, *, relu):
    cnt = (pc_ref[0] + pc_ref[1])[:, 0:1]
    inv = 1.0 / jnp.maximum(cnt, 1.0)
    agg = (p_ref[0] + p_ref[1]) * inv
    out = lax.dot_general(agg, wl_ref[...], (((1,), (1,)), ((), ())),
                          preferred_element_type=jnp.float32)
    out = out + lax.dot_general(x_ref[...], wr_ref[...],
                                (((1,), (1,)), ((), ())),
                                preferred_element_type=jnp.float32)
    out = out + bl_ref[...]
    if relu:
        out = jnp.maximum(out, 0.0)
    o_ref[...] = out


def _tc_layer(p, pc, x, wl, wr, bl2, relu):
    return pl.pallas_call(
        functools.partial(_tc_body, relu=relu),
        grid=(N2 // BLK,),
        in_specs=[
            pl.BlockSpec((NC, BLK, H), lambda i: (0, i, 0)),
            pl.BlockSpec((NC, BLK, H), lambda i: (0, i, 0)),
            pl.BlockSpec((BLK, H), lambda i: (i, 0)),
            pl.BlockSpec((H, H), lambda i: (0, 0)),
            pl.BlockSpec((H, H), lambda i: (0, 0)),
            pl.BlockSpec((1, H), lambda i: (0, 0)),
        ],
        out_specs=pl.BlockSpec((BLK, H), lambda i: (i, 0)),
        out_shape=jax.ShapeDtypeStruct((N2, H), jnp.float32),
    )(p, pc, x, wl, wr, bl2)


def kernel(node_id, edge_index, emb, Wl1, bl1, Wr1, Wl2, bl2, Wr2,
           Wl3, bl3, Wr3):
    src = edge_index[0]
    dst = edge_index[1]
    src_f = jnp.concatenate([src, jnp.zeros((EPAD - E,), jnp.int32)])
    dst_f = jnp.concatenate([dst, jnp.full((EPAD - E,), N, jnp.int32)])
    s0 = src_f[:E0].reshape(NS, CH_CORE[0], CS)
    d0 = dst_f[:E0].reshape(NS, CH_CORE[0], CS)
    s1 = src_f[E0:].reshape(NS, CH_CORE[1], CS)
    d1 = dst_f[E0:].reshape(NS, CH_CORE[1], CS)
    nid_p = jnp.concatenate(
        [node_id, jnp.zeros((N2 - N,), jnp.int32)]).reshape(NC, NS, 5, 64)
    zr = jnp.zeros((RPS, H), jnp.float32)
    ones = jnp.ones((CS, H), jnp.float32)

    dc = dst_f.reshape(NC, NS, CH_CNT[0], CS)
    x0, pc = _prep_kernel(emb, nid_p, dc[0], dc[1], zr, ones)

    x = x0
    for wl, bl, wr, relu in ((Wl1, bl1, Wr1, True),
                             (Wl2, bl2, Wr2, True),
                             (Wl3, bl3, Wr3, False)):
        p = _seg_kernel(x, s0, d0, s1, d1, zr)
        x = _tc_layer(p, pc, x, wl, wr, bl.reshape(1, H), relu)
    return x[:N]


# speedup vs baseline: 1.2023x; 1.2023x over previous
import functools

import jax
import jax.numpy as jnp
from jax import lax
from jax.experimental import pallas as pl
from jax.experimental.pallas import tpu as pltpu
from jax.experimental.pallas import tpu_sc as plsc

N = 10000
H = 128
E = 320000

NC = 2
NS = 16
NW = NC * NS

N2 = 10240
RPW = N2 // NW
RPS = N2 // NS
CS = 128
CH_CORE = (120, 40)
CHMAX = max(CH_CORE)
E0 = NS * CH_CORE[0] * CS
E1 = NS * CH_CORE[1] * CS
EPAD = E0 + E1

_mesh = plsc.VectorSubcoreMesh(
    core_axis_name="c", subcore_axis_name="s", num_cores=NC, num_subcores=NS)


CH_CNT = (EPAD // (2 * NS * CS), EPAD // (2 * NS * CS))


def _prep_body(emb_h, nid_h, d0_h, d1_h, zr_h, ones_h, x0_h, pc_h,
               nid_v, rows_v, didx_v, ones_v, cnt_sh):
    c = lax.axis_index("c")
    s = lax.axis_index("s")
    pltpu.sync_copy(nid_h.at[c, s], nid_v)
    pltpu.sync_copy(ones_h, ones_v)
    pltpu.sync_copy(zr_h, cnt_sh.at[pl.ds(s * RPS, RPS)])

    @pl.when(c == 0)
    def _():
        pltpu.sync_copy(d0_h.at[s], didx_v.at[pl.ds(0, CH_CNT[0])])

    @pl.when(c == 1)
    def _():
        pltpu.sync_copy(d1_h.at[s], didx_v.at[pl.ds(0, CH_CNT[1])])

    plsc.subcore_barrier()
    w = c * NS + s
    for j in range(5):
        pltpu.sync_copy(emb_h.at[nid_v.at[j]], rows_v)
        pltpu.sync_copy(rows_v, x0_h.at[pl.ds(w * RPW + j * 64, 64)])

    def count_loop(nch):
        def body(k, _):
            pltpu.sync_copy(ones_v, cnt_sh.at[didx_v.at[k]], add=True)
            return 0
        lax.fori_loop(0, nch, body, 0)

    @pl.when(c == 0)
    def _():
        count_loop(CH_CNT[0])

    @pl.when(c == 1)
    def _():
        count_loop(CH_CNT[1])

    plsc.subcore_barrier()
    pltpu.sync_copy(cnt_sh.at[pl.ds(s * RPS, RPS)],
                    pc_h.at[c, pl.ds(s * RPS, RPS)])


_prep_kernel = pl.kernel(
    _prep_body,
    out_type=(jax.ShapeDtypeStruct((N2, H), jnp.float32),
              jax.ShapeDtypeStruct((NC, N2, H), jnp.float32)),
    mesh=_mesh,
    scratch_types=[
        pltpu.VMEM((5, 64), jnp.int32),
        pltpu.VMEM((64, H), jnp.float32),
        pltpu.VMEM((CHMAX, CS), jnp.int32),
        pltpu.VMEM((CS, H), jnp.float32),
        pltpu.VMEM_SHARED((N2, H), jnp.float32),
    ],
)


def _seg_body(x_h, s0_h, d0_h, s1_h, d1_h, zr_h, p_h,
              sidx_v, didx_v, rows_v, acc_sh):
    c = lax.axis_index("c")
    s = lax.axis_index("s")
    pltpu.sync_copy(zr_h, acc_sh.at[pl.ds(s * RPS, RPS)])

    @pl.when(c == 0)
    def _():
        pltpu.sync_copy(s0_h.at[s], sidx_v.at[pl.ds(0, CH_CORE[0])])
        pltpu.sync_copy(d0_h.at[s], didx_v.at[pl.ds(0, CH_CORE[0])])

    @pl.when(c == 1)
    def _():
        pltpu.sync_copy(s1_h.at[s], sidx_v.at[pl.ds(0, CH_CORE[1])])
        pltpu.sync_copy(d1_h.at[s], didx_v.at[pl.ds(0, CH_CORE[1])])

    plsc.subcore_barrier()

    def edge_loop(nch):
        def body(k, _):
            pltpu.sync_copy(x_h.at[sidx_v.at[k]], rows_v)
            pltpu.sync_copy(rows_v, acc_sh.at[didx_v.at[k]], add=True)
            return 0
        lax.fori_loop(0, nch, body, 0)

    @pl.when(c == 0)
    def _():
        edge_loop(CH_CORE[0])

    @pl.when(c == 1)
    def _():
        edge_loop(CH_CORE[1])

    plsc.subcore_barrier()
    pltpu.sync_copy(acc_sh.at[pl.ds(s * RPS, RPS)],
                    p_h.at[c, pl.ds(s * RPS, RPS)])


_seg_kernel = pl.kernel(
    _seg_body,
    out_type=jax.ShapeDtypeStruct((NC, N2, H), jnp.float32),
    mesh=_mesh,
    scratch_types=[
        pltpu.VMEM((CHMAX, CS), jnp.int32),
        pltpu.VMEM((CHMAX, CS), jnp.int32),
        pltpu.VMEM((CS, H), jnp.float32),
        pltpu.VMEM_SHARED((N2, H), jnp.float32),
    ],
)

BLK = 1280


def _tc_body(p_ref, pc_ref, x_ref, wl_ref, wr_ref, bl_ref, o_ref, *, relu):
    cnt = (pc_ref[0] + pc_ref[1])[:, 0:1]
    inv = 1.0 / jnp.maximum(cnt, 1.0)
    agg = (p_ref[0] + p_ref[1]) * inv
    out = lax.dot_general(agg, wl_ref[...], (((1,), (1,)), ((), ())),
                          preferred_element_type=jnp.float32)
    out = out + lax.dot_general(x_ref[...], wr_ref[...],
                                (((1,), (1,)), ((), ())),
                                preferred_element_type=jnp.float32)
    out = out + bl_ref[...]
    if relu:
        out = jnp.maximum(out, 0.0)
    o_ref[...] = out


def _tc_layer(p, pc, x, wl, wr, bl2, relu):
    return pl.pallas_call(
        functools.partial(_tc_body, relu=relu),
        grid=(N2 // BLK,),
        in_specs=[
            pl.BlockSpec((NC, BLK, H), lambda i: (0, i, 0)),
            pl.BlockSpec((NC, BLK, H), lambda i: (0, i, 0)),
            pl.BlockSpec((BLK, H), lambda i: (i, 0)),
            pl.BlockSpec((H, H), lambda i: (0, 0)),
            pl.BlockSpec((H, H), lambda i: (0, 0)),
            pl.BlockSpec((1, H), lambda i: (0, 0)),
        ],
        out_specs=pl.BlockSpec((BLK, H), lambda i: (i, 0)),
        out_shape=jax.ShapeDtypeStruct((N2, H), jnp.float32),
    )(p, pc, x, wl, wr, bl2)


def kernel(node_id, edge_index, emb, Wl1, bl1, Wr1, Wl2, bl2, Wr2,
           Wl3, bl3, Wr3):
    src = edge_index[0]
    dst = edge_index[1]
    src_f = jnp.concatenate([src, jnp.zeros((EPAD - E,), jnp.int32)])
    dst_f = jnp.concatenate([dst, jnp.full((EPAD - E,), N, jnp.int32)])
    s0 = src_f[:E0].reshape(NS, CH_CORE[0], CS)
    d0 = dst_f[:E0].reshape(NS, CH_CORE[0], CS)
    s1 = src_f[E0:].reshape(NS, CH_CORE[1], CS)
    d1 = dst_f[E0:].reshape(NS, CH_CORE[1], CS)
    nid_p = jnp.concatenate(
        [node_id, jnp.zeros((N2 - N,), jnp.int32)]).reshape(NC, NS, 5, 64)
    zr = jnp.zeros((RPS, H), jnp.float32)
    ones = jnp.ones((CS, H), jnp.float32)

    dc = dst_f.reshape(NC, NS, CH_CNT[0], CS)
    x0, pc = _prep_kernel(emb, nid_p, dc[0], dc[1], zr, ones)

    x = x0
    for wl, bl, wr, relu in ((Wl1, bl1, Wr1, True),
                             (Wl2, bl2, Wr2, True),
                             (Wl3, bl3, Wr3, False)):
        p = _seg_kernel(x, s0, d0, s1, d1, zr)
        x = _tc_layer(p, pc, x, wl, wr, bl.reshape(1, H), relu)
    return x[:N]
